# baseline (speedup 1.0000x reference)
import functools

import jax
import jax.numpy as jnp
from jax import lax
from jax.experimental import pallas as pl
from jax.experimental.pallas import tpu as pltpu
from jax.experimental.pallas import tpu_sc as plsc

N = 10000
E = 320000
D = 128
NC = 2
NS = 16
EPT = E // NS
K = 100
NCHUNK = EPT // K
G = 10
NG = NCHUNK // G
STRIPE = 640
STRIPE_LAST = N - (NS - 1) * STRIPE

_MESH = plsc.VectorSubcoreMesh(core_axis_name="c", subcore_axis_name="s")


@functools.partial(
    pl.kernel,
    out_type=jax.ShapeDtypeStruct((NC * NS * N,), jnp.float32),
    mesh=_MESH,
    scratch_types=[
        pltpu.VMEM((EPT,), jnp.int32),
        pltpu.VMEM((N,), jnp.float32),
    ],
    compiler_params=pltpu.CompilerParams(needs_layout_passes=False),
)
def _deg_kernel(dst_hbm, deg_out, idx_v, deg_local):
    c = lax.axis_index("c")
    s = lax.axis_index("s")
    wid = c * NS + s

    def zbody(i, carry):
        deg_local[pl.ds(i * 16, 16)] = jnp.zeros((16,), jnp.float32)
        return carry

    lax.fori_loop(0, N // 16, zbody, 0)

    pltpu.sync_copy(dst_hbm.at[pl.ds(wid * EPT, EPT)], idx_v)
    ones16 = jnp.full((16,), 1.0, dtype=jnp.float32)

    def body(i, carry):
        idx16 = idx_v[pl.ds(i * 16, 16)]
        plsc.addupdate_scatter(deg_local, [idx16], ones16)
        return carry

    lax.fori_loop(0, EPT // 16, body, 0)
    pltpu.sync_copy(deg_local, deg_out.at[pl.ds(wid * N, N)])


@functools.partial(
    pl.kernel,
    out_type=jax.ShapeDtypeStruct((NC * N, D), jnp.float32),
    mesh=_MESH,
    scratch_types=[
        pltpu.VMEM((G, K), jnp.int32),
        pltpu.VMEM((G, K), jnp.int32),
        pltpu.VMEM((K, D), jnp.float32),
        pltpu.VMEM((K, D), jnp.float32),
        pltpu.VMEM_SHARED((N, D), jnp.float32),
        pltpu.SemaphoreType.DMA,
        pltpu.SemaphoreType.DMA,
        pltpu.SemaphoreType.DMA,
        pltpu.SemaphoreType.DMA,
    ],
    compiler_params=pltpu.CompilerParams(needs_layout_passes=False),
)
def _scatter_kernel(z_hbm, src_hbm, dst_hbm, acc_out, src_v, dst_v, gbuf0,
                    gbuf1, acc_sp, sg0, sg1, ss0, ss1):
    c = lax.axis_index("c")
    s = lax.axis_index("s")
    wid = c * NS + s

    @pl.when(s < NS - 1)
    def _():
        pltpu.sync_copy(z_hbm.at[pl.ds(c * N + s * STRIPE, STRIPE)],
                        acc_sp.at[pl.ds(s * STRIPE, STRIPE)])

    @pl.when(s == NS - 1)
    def _():
        pltpu.sync_copy(
            z_hbm.at[pl.ds(c * N + (NS - 1) * STRIPE, STRIPE_LAST)],
            acc_sp.at[pl.ds((NS - 1) * STRIPE, STRIPE_LAST)])

    plsc.subcore_barrier()

    def group_body(gi, carry):
        pltpu.sync_copy(src_hbm.at[wid, gi], src_v)
        pltpu.sync_copy(dst_hbm.at[wid, gi], dst_v)
        pltpu.async_copy(z_hbm.at[src_v.at[0]], gbuf0, sg0)

        def body(p, carry2):
            j0 = 2 * p

            @pl.when(p > 0)
            def _():
                pltpu.make_async_copy(gbuf1, acc_sp.at[dst_v.at[j0 - 1]],
                                      ss1).wait()

            pltpu.async_copy(z_hbm.at[src_v.at[j0 + 1]], gbuf1, sg1)
            pltpu.make_async_copy(z_hbm.at[src_v.at[j0]], gbuf0, sg0).wait()
            pltpu.async_copy(gbuf0, acc_sp.at[dst_v.at[j0]], ss0, add=True)
            pltpu.make_async_copy(z_hbm.at[src_v.at[j0 + 1]], gbuf1,
                                  sg1).wait()
            pltpu.async_copy(gbuf1, acc_sp.at[dst_v.at[j0 + 1]], ss1,
                             add=True)
            pltpu.make_async_copy(gbuf0, acc_sp.at[dst_v.at[j0]], ss0).wait()

            @pl.when(p < G // 2 - 1)
            def _():
                pltpu.async_copy(z_hbm.at[src_v.at[j0 + 2]], gbuf0, sg0)

            return carry2

        lax.fori_loop(0, G // 2, body, 0)
        pltpu.make_async_copy(gbuf1, acc_sp.at[dst_v.at[G - 1]], ss1).wait()
        return carry

    lax.fori_loop(0, NG, group_body, 0)
    plsc.subcore_barrier()

    @pl.when(s < NS - 1)
    def _():
        pltpu.sync_copy(acc_sp.at[pl.ds(s * STRIPE, STRIPE)],
                        acc_out.at[pl.ds(c * N + s * STRIPE, STRIPE)])

    @pl.when(s == NS - 1)
    def _():
        pltpu.sync_copy(
            acc_sp.at[pl.ds((NS - 1) * STRIPE, STRIPE_LAST)],
            acc_out.at[pl.ds(c * N + (NS - 1) * STRIPE, STRIPE_LAST)])


def _dinv_body(degs_ref, dinv_ref):
    d = jnp.sum(degs_ref[0], axis=0, keepdims=True)
    dinv_ref[0] = lax.rsqrt(d + 1.0)


def _z1_body(x_ref, w_ref, dinv_ref, z_ref):
    xw = jnp.dot(x_ref[0], w_ref[...], preferred_element_type=jnp.float32)
    z_ref[0] = xw * dinv_ref[0]


def _z2_body(acc_ref, dinv_ref, b1_ref, w2_ref, z2_ref):
    dinv = dinv_ref[0]
    h = jnp.maximum(acc_ref[0] * dinv + b1_ref[...], 0.0)
    z2_ref[0] = jnp.dot(h, w2_ref[...],
                        preferred_element_type=jnp.float32) * dinv


def _out_body(acc_ref, dinv_ref, b2_ref, fcw_ref, fcb_ref, o_ref):
    g = jnp.maximum(acc_ref[0] * dinv_ref[0] + b2_ref[...], 0.0)
    pooled = jnp.sum(g, axis=0, keepdims=True) * (1.0 / N)
    o_ref[0] = jnp.dot(pooled, fcw_ref[...],
                       preferred_element_type=jnp.float32) + fcb_ref[...]


def _b_spec(shape):
    nd = len(shape)
    return pl.BlockSpec(shape, lambda b: (0,) * nd)


def _row_spec(shape):
    nd = len(shape)
    return pl.BlockSpec((1,) + shape[1:], lambda b: (b,) + (0,) * (nd - 1))


def kernel(x1, edge_index1, x2, edge_index2, W1, b1, W2, b2, fcW, fcb):
    src = jnp.concatenate([edge_index1[0], edge_index2[0] + N])
    dst = jnp.concatenate([edge_index1[1], edge_index2[1]])
    src3 = src.reshape(NC * NS, NG, G, K)
    dst3 = dst.reshape(NC * NS, NG, G, K)
    X = jnp.stack([x1, x2])
    b1r = b1.reshape(1, D)
    b2r = b2.reshape(1, D)
    fcbr = fcb.reshape(1, D)

    degs = _deg_kernel(dst).reshape(NC, NS, N)

    dinv = pl.pallas_call(
        _dinv_body,
        grid=(NC,),
        in_specs=[_row_spec((NC, NS, N))],
        out_specs=_row_spec((NC, 1, N)),
        out_shape=jax.ShapeDtypeStruct((NC, 1, N), jnp.float32),
    )(degs)
    dinv_col = dinv.reshape(NC, N, 1)

    z1 = pl.pallas_call(
        _z1_body,
        grid=(NC,),
        in_specs=[_row_spec((NC, N, D)), _b_spec((D, D)),
                  _row_spec((NC, N, 1))],
        out_specs=_row_spec((NC, N, D)),
        out_shape=jax.ShapeDtypeStruct((NC, N, D), jnp.float32),
    )(X, W1, dinv_col)

    acc1 = _scatter_kernel(z1.reshape(NC * N, D), src3, dst3)

    z2 = pl.pallas_call(
        _z2_body,
        grid=(NC,),
        in_specs=[_row_spec((NC, N, D)), _row_spec((NC, N, 1)),
                  _b_spec((1, D)), _b_spec((D, D))],
        out_specs=_row_spec((NC, N, D)),
        out_shape=jax.ShapeDtypeStruct((NC, N, D), jnp.float32),
    )(acc1.reshape(NC, N, D), dinv_col, b1r, W2)

    acc2 = _scatter_kernel(z2.reshape(NC * N, D), src3, dst3)

    out = pl.pallas_call(
        _out_body,
        grid=(NC,),
        in_specs=[_row_spec((NC, N, D)), _row_spec((NC, N, 1)),
                  _b_spec((1, D)), _b_spec((D, D)), _b_spec((1, D))],
        out_specs=_row_spec((NC, 1, D)),
        out_shape=jax.ShapeDtypeStruct((NC, 1, D), jnp.float32),
    )(acc2.reshape(NC, N, D), dinv_col, b2r, fcW, fcbr)

    return (out[0, 0], out[1, 0])


# speedup vs baseline: 1.2111x; 1.2111x over previous
import functools

import jax
import jax.numpy as jnp
from jax import lax
from jax.experimental import pallas as pl
from jax.experimental.pallas import tpu as pltpu
from jax.experimental.pallas import tpu_sc as plsc

N = 10000
E = 320000
D = 128
NC = 2
NS = 16
EPT = E // NS
K = 100
NCHUNK = EPT // K
G = 10
NG = NCHUNK // G
STRIPE = 640
STRIPE_LAST = N - (NS - 1) * STRIPE

_MESH = plsc.VectorSubcoreMesh(core_axis_name="c", subcore_axis_name="s")


@functools.partial(
    pl.kernel,
    out_type=jax.ShapeDtypeStruct((NC * NS * N,), jnp.float32),
    mesh=_MESH,
    scratch_types=[
        pltpu.VMEM((EPT,), jnp.int32),
        pltpu.VMEM((N,), jnp.float32),
    ],
    compiler_params=pltpu.CompilerParams(needs_layout_passes=False),
)
def _deg_kernel(dst_hbm, deg_out, idx_v, deg_local):
    c = lax.axis_index("c")
    s = lax.axis_index("s")
    wid = c * NS + s

    def zbody(i, carry):
        deg_local[pl.ds(i * 16, 16)] = jnp.zeros((16,), jnp.float32)
        return carry

    lax.fori_loop(0, N // 16, zbody, 0)

    pltpu.sync_copy(dst_hbm.at[pl.ds(wid * EPT, EPT)], idx_v)
    ones16 = jnp.full((16,), 1.0, dtype=jnp.float32)

    def body(i, carry):
        idx16 = idx_v[pl.ds(i * 16, 16)]
        plsc.addupdate_scatter(deg_local, [idx16], ones16)
        return carry

    lax.fori_loop(0, EPT // 16, body, 0)
    pltpu.sync_copy(deg_local, deg_out.at[pl.ds(wid * N, N)])


@functools.partial(
    pl.kernel,
    out_type=jax.ShapeDtypeStruct((NC * N, D), jnp.float32),
    mesh=_MESH,
    scratch_types=[
        pltpu.VMEM((G, K), jnp.int32),
        pltpu.VMEM((G, K), jnp.int32),
        pltpu.VMEM((K, D), jnp.float32),
        pltpu.VMEM((K, D), jnp.float32),
        pltpu.VMEM_SHARED((N, D), jnp.float32),
        pltpu.SemaphoreType.DMA,
        pltpu.SemaphoreType.DMA,
    ],
    compiler_params=pltpu.CompilerParams(needs_layout_passes=False),
)
def _scatter_kernel(z_hbm, src_hbm, dst_hbm, acc_out, src_v, dst_v, gbuf0,
                    gbuf1, acc_sp, sg0, sg1):
    c = lax.axis_index("c")
    s = lax.axis_index("s")
    wid = c * NS + s

    @pl.when(s < NS - 1)
    def _():
        pltpu.sync_copy(z_hbm.at[pl.ds(c * N + s * STRIPE, STRIPE)],
                        acc_sp.at[pl.ds(s * STRIPE, STRIPE)])

    @pl.when(s == NS - 1)
    def _():
        pltpu.sync_copy(
            z_hbm.at[pl.ds(c * N + (NS - 1) * STRIPE, STRIPE_LAST)],
            acc_sp.at[pl.ds((NS - 1) * STRIPE, STRIPE_LAST)])

    plsc.subcore_barrier()

    def group_body(gi, carry):
        pltpu.sync_copy(src_hbm.at[wid, gi], src_v)
        pltpu.sync_copy(dst_hbm.at[wid, gi], dst_v)
        pltpu.async_copy(z_hbm.at[src_v.at[0]], gbuf0, sg0)

        def body(p, carry2):
            j0 = 2 * p
            pltpu.async_copy(z_hbm.at[src_v.at[j0 + 1]], gbuf1, sg1)
            pltpu.make_async_copy(z_hbm.at[src_v.at[j0]], gbuf0, sg0).wait()
            pltpu.sync_copy(gbuf0, acc_sp.at[dst_v.at[j0]], add=True)

            @pl.when(p < G // 2 - 1)
            def _():
                pltpu.async_copy(z_hbm.at[src_v.at[j0 + 2]], gbuf0, sg0)

            pltpu.make_async_copy(z_hbm.at[src_v.at[j0 + 1]], gbuf1,
                                  sg1).wait()
            pltpu.sync_copy(gbuf1, acc_sp.at[dst_v.at[j0 + 1]], add=True)
            return carry2

        lax.fori_loop(0, G // 2, body, 0)
        return carry

    lax.fori_loop(0, NG, group_body, 0)
    plsc.subcore_barrier()

    @pl.when(s < NS - 1)
    def _():
        pltpu.sync_copy(acc_sp.at[pl.ds(s * STRIPE, STRIPE)],
                        acc_out.at[pl.ds(c * N + s * STRIPE, STRIPE)])

    @pl.when(s == NS - 1)
    def _():
        pltpu.sync_copy(
            acc_sp.at[pl.ds((NS - 1) * STRIPE, STRIPE_LAST)],
            acc_out.at[pl.ds(c * N + (NS - 1) * STRIPE, STRIPE_LAST)])


def _z1_body(x_ref, w_ref, degs_ref, z_ref, dinv_ref):
    d = jnp.sum(degs_ref[0], axis=0, keepdims=True)
    dinv_col = jnp.transpose(lax.rsqrt(d + 1.0))
    dinv_ref[0] = dinv_col
    xw = jnp.dot(x_ref[0], w_ref[...], preferred_element_type=jnp.float32)
    z_ref[0] = xw * dinv_col


def _z2_body(acc_ref, dinv_ref, b1_ref, w2_ref, z2_ref):
    dinv = dinv_ref[0]
    h = jnp.maximum(acc_ref[0] * dinv + b1_ref[...], 0.0)
    z2_ref[0] = jnp.dot(h, w2_ref[...],
                        preferred_element_type=jnp.float32) * dinv


def _out_body(acc_ref, dinv_ref, b2_ref, fcw_ref, fcb_ref, o_ref):
    g = jnp.maximum(acc_ref[0] * dinv_ref[0] + b2_ref[...], 0.0)
    pooled = jnp.sum(g, axis=0, keepdims=True) * (1.0 / N)
    o_ref[0] = jnp.dot(pooled, fcw_ref[...],
                       preferred_element_type=jnp.float32) + fcb_ref[...]


def _b_spec(shape):
    nd = len(shape)
    return pl.BlockSpec(shape, lambda b: (0,) * nd)


def _row_spec(shape):
    nd = len(shape)
    return pl.BlockSpec((1,) + shape[1:], lambda b: (b,) + (0,) * (nd - 1))


def kernel(x1, edge_index1, x2, edge_index2, W1, b1, W2, b2, fcW, fcb):
    src = jnp.concatenate([edge_index1[0], edge_index2[0] + N])
    dst = jnp.concatenate([edge_index1[1], edge_index2[1]])
    src3 = src.reshape(NC * NS, NG, G, K)
    dst3 = dst.reshape(NC * NS, NG, G, K)
    X = jnp.stack([x1, x2])
    b1r = b1.reshape(1, D)
    b2r = b2.reshape(1, D)
    fcbr = fcb.reshape(1, D)

    degs = _deg_kernel(dst).reshape(NC, NS, N)

    z1, dinv_col = pl.pallas_call(
        _z1_body,
        grid=(NC,),
        in_specs=[_row_spec((NC, N, D)), _b_spec((D, D)),
                  _row_spec((NC, NS, N))],
        out_specs=[_row_spec((NC, N, D)), _row_spec((NC, N, 1))],
        out_shape=[jax.ShapeDtypeStruct((NC, N, D), jnp.float32),
                   jax.ShapeDtypeStruct((NC, N, 1), jnp.float32)],
    )(X, W1, degs)

    acc1 = _scatter_kernel(z1.reshape(NC * N, D), src3, dst3)

    z2 = pl.pallas_call(
        _z2_body,
        grid=(NC,),
        in_specs=[_row_spec((NC, N, D)), _row_spec((NC, N, 1)),
                  _b_spec((1, D)), _b_spec((D, D))],
        out_specs=_row_spec((NC, N, D)),
        out_shape=jax.ShapeDtypeStruct((NC, N, D), jnp.float32),
    )(acc1.reshape(NC, N, D), dinv_col, b1r, W2)

    acc2 = _scatter_kernel(z2.reshape(NC * N, D), src3, dst3)

    out = pl.pallas_call(
        _out_body,
        grid=(NC,),
        in_specs=[_row_spec((NC, N, D)), _row_spec((NC, N, 1)),
                  _b_spec((1, D)), _b_spec((D, D)), _b_spec((1, D))],
        out_specs=_row_spec((NC, 1, D)),
        out_shape=jax.ShapeDtypeStruct((NC, 1, D), jnp.float32),
    )(acc2.reshape(NC, N, D), dinv_col, b2r, fcW, fcbr)

    return (out[0, 0], out[1, 0])


# speedup vs baseline: 1.3098x; 1.0815x over previous
import functools

import jax
import jax.numpy as jnp
from jax import lax
from jax.experimental import pallas as pl
from jax.experimental.pallas import tpu as pltpu
from jax.experimental.pallas import tpu_sc as plsc

N = 10000
E = 320000
D = 128
NC = 2
NS = 16
EPT = E // NS
K = 100
NCHUNK = EPT // K
G = 10
NG = NCHUNK // G
STRIPE = 640
STRIPE_LAST = N - (NS - 1) * STRIPE

_MESH = plsc.VectorSubcoreMesh(core_axis_name="c", subcore_axis_name="s")


@functools.partial(
    pl.kernel,
    out_type=jax.ShapeDtypeStruct((NC * NS * N,), jnp.float32),
    mesh=_MESH,
    scratch_types=[
        pltpu.VMEM((EPT,), jnp.int32),
        pltpu.VMEM((N,), jnp.float32),
    ],
    compiler_params=pltpu.CompilerParams(needs_layout_passes=False),
)
def _deg_kernel(dst_hbm, deg_out, idx_v, deg_local):
    c = lax.axis_index("c")
    s = lax.axis_index("s")
    wid = c * NS + s

    def zbody(i, carry):
        deg_local[pl.ds(i * 16, 16)] = jnp.zeros((16,), jnp.float32)
        return carry

    lax.fori_loop(0, N // 16, zbody, 0)

    pltpu.sync_copy(dst_hbm.at[pl.ds(wid * EPT, EPT)], idx_v)
    ones16 = jnp.full((16,), 1.0, dtype=jnp.float32)

    def body(i, carry):
        idx16 = idx_v[pl.ds(i * 16, 16)]
        plsc.addupdate_scatter(deg_local, [idx16], ones16)
        return carry

    lax.fori_loop(0, EPT // 16, body, 0)
    pltpu.sync_copy(deg_local, deg_out.at[pl.ds(wid * N, N)])


@functools.partial(
    pl.kernel,
    out_type=jax.ShapeDtypeStruct((NC * N, D), jnp.float32),
    mesh=_MESH,
    scratch_types=[
        pltpu.VMEM((2, G, K), jnp.int32),
        pltpu.VMEM((2, G, K), jnp.int32),
        pltpu.VMEM((K, D), jnp.float32),
        pltpu.VMEM((K, D), jnp.float32),
        pltpu.VMEM_SHARED((N, D), jnp.float32),
        pltpu.SemaphoreType.DMA,
        pltpu.SemaphoreType.DMA,
        pltpu.SemaphoreType.DMA,
    ],
    compiler_params=pltpu.CompilerParams(needs_layout_passes=False),
)
def _scatter_kernel(z_hbm, src_hbm, dst_hbm, acc_out, src_v, dst_v, gbuf0,
                    gbuf1, acc_sp, sg0, sg1, si):
    c = lax.axis_index("c")
    s = lax.axis_index("s")
    wid = c * NS + s

    @pl.when(s < NS - 1)
    def _():
        pltpu.sync_copy(z_hbm.at[pl.ds(c * N + s * STRIPE, STRIPE)],
                        acc_sp.at[pl.ds(s * STRIPE, STRIPE)])

    @pl.when(s == NS - 1)
    def _():
        pltpu.sync_copy(
            z_hbm.at[pl.ds(c * N + (NS - 1) * STRIPE, STRIPE_LAST)],
            acc_sp.at[pl.ds((NS - 1) * STRIPE, STRIPE_LAST)])

    plsc.subcore_barrier()

    pltpu.sync_copy(src_hbm.at[wid, 0], src_v.at[0])
    pltpu.sync_copy(dst_hbm.at[wid, 0], dst_v.at[0])

    def group_body(gi, carry):
        b = lax.rem(gi, 2)
        sv = src_v.at[b]
        dv = dst_v.at[b]

        @pl.when(gi < NG - 1)
        def _():
            pltpu.async_copy(src_hbm.at[wid, gi + 1], src_v.at[1 - b], si)
            pltpu.async_copy(dst_hbm.at[wid, gi + 1], dst_v.at[1 - b], si)

        pltpu.async_copy(z_hbm.at[sv.at[0]], gbuf0, sg0)

        def body(p, carry2):
            j0 = 2 * p
            pltpu.async_copy(z_hbm.at[sv.at[j0 + 1]], gbuf1, sg1)
            pltpu.make_async_copy(z_hbm.at[sv.at[j0]], gbuf0, sg0).wait()
            pltpu.sync_copy(gbuf0, acc_sp.at[dv.at[j0]], add=True)

            @pl.when(p < G // 2 - 1)
            def _():
                pltpu.async_copy(z_hbm.at[sv.at[j0 + 2]], gbuf0, sg0)

            pltpu.make_async_copy(z_hbm.at[sv.at[j0 + 1]], gbuf1,
                                  sg1).wait()
            pltpu.sync_copy(gbuf1, acc_sp.at[dv.at[j0 + 1]], add=True)
            return carry2

        lax.fori_loop(0, G // 2, body, 0)

        @pl.when(gi < NG - 1)
        def _():
            pltpu.make_async_copy(src_hbm.at[wid, gi + 1], src_v.at[1 - b],
                                  si).wait()
            pltpu.make_async_copy(dst_hbm.at[wid, gi + 1], dst_v.at[1 - b],
                                  si).wait()

        return carry

    lax.fori_loop(0, NG, group_body, 0)
    plsc.subcore_barrier()

    @pl.when(s < NS - 1)
    def _():
        pltpu.sync_copy(acc_sp.at[pl.ds(s * STRIPE, STRIPE)],
                        acc_out.at[pl.ds(c * N + s * STRIPE, STRIPE)])

    @pl.when(s == NS - 1)
    def _():
        pltpu.sync_copy(
            acc_sp.at[pl.ds((NS - 1) * STRIPE, STRIPE_LAST)],
            acc_out.at[pl.ds(c * N + (NS - 1) * STRIPE, STRIPE_LAST)])


def _z1_body(x_ref, w_ref, degs_ref, z_ref, dinv_ref):
    d = jnp.sum(degs_ref[0], axis=0, keepdims=True)
    dinv_col = jnp.transpose(lax.rsqrt(d + 1.0))
    dinv_ref[0] = dinv_col
    xw = jnp.dot(x_ref[0], w_ref[...], preferred_element_type=jnp.float32)
    z_ref[0] = xw * dinv_col


def _z2_body(acc_ref, dinv_ref, b1_ref, w2_ref, z2_ref):
    dinv = dinv_ref[0]
    h = jnp.maximum(acc_ref[0] * dinv + b1_ref[...], 0.0)
    z2_ref[0] = jnp.dot(h, w2_ref[...],
                        preferred_element_type=jnp.float32) * dinv


def _out_body(acc_ref, dinv_ref, b2_ref, fcw_ref, fcb_ref, o_ref):
    g = jnp.maximum(acc_ref[0] * dinv_ref[0] + b2_ref[...], 0.0)
    pooled = jnp.sum(g, axis=0, keepdims=True) * (1.0 / N)
    o_ref[0] = jnp.dot(pooled, fcw_ref[...],
                       preferred_element_type=jnp.float32) + fcb_ref[...]


def _b_spec(shape):
    nd = len(shape)
    return pl.BlockSpec(shape, lambda b: (0,) * nd)


def _row_spec(shape):
    nd = len(shape)
    return pl.BlockSpec((1,) + shape[1:], lambda b: (b,) + (0,) * (nd - 1))


def kernel(x1, edge_index1, x2, edge_index2, W1, b1, W2, b2, fcW, fcb):
    src = jnp.concatenate([edge_index1[0], edge_index2[0] + N])
    dst = jnp.concatenate([edge_index1[1], edge_index2[1]])
    src3 = src.reshape(NC * NS, NG, G, K)
    dst3 = dst.reshape(NC * NS, NG, G, K)
    X = jnp.stack([x1, x2])
    b1r = b1.reshape(1, D)
    b2r = b2.reshape(1, D)
    fcbr = fcb.reshape(1, D)

    degs = _deg_kernel(dst).reshape(NC, NS, N)

    z1, dinv_col = pl.pallas_call(
        _z1_body,
        grid=(NC,),
        in_specs=[_row_spec((NC, N, D)), _b_spec((D, D)),
                  _row_spec((NC, NS, N))],
        out_specs=[_row_spec((NC, N, D)), _row_spec((NC, N, 1))],
        out_shape=[jax.ShapeDtypeStruct((NC, N, D), jnp.float32),
                   jax.ShapeDtypeStruct((NC, N, 1), jnp.float32)],
    )(X, W1, degs)

    acc1 = _scatter_kernel(z1.reshape(NC * N, D), src3, dst3)

    z2 = pl.pallas_call(
        _z2_body,
        grid=(NC,),
        in_specs=[_row_spec((NC, N, D)), _row_spec((NC, N, 1)),
                  _b_spec((1, D)), _b_spec((D, D))],
        out_specs=_row_spec((NC, N, D)),
        out_shape=jax.ShapeDtypeStruct((NC, N, D), jnp.float32),
    )(acc1.reshape(NC, N, D), dinv_col, b1r, W2)

    acc2 = _scatter_kernel(z2.reshape(NC * N, D), src3, dst3)

    out = pl.pallas_call(
        _out_body,
        grid=(NC,),
        in_specs=[_row_spec((NC, N, D)), _row_spec((NC, N, 1)),
                  _b_spec((1, D)), _b_spec((D, D)), _b_spec((1, D))],
        out_specs=_row_spec((NC, 1, D)),
        out_shape=jax.ShapeDtypeStruct((NC, 1, D), jnp.float32),
    )(acc2.reshape(NC, N, D), dinv_col, b2r, fcW, fcbr)

    return (out[0, 0], out[1, 0])
